# baseline (speedup 1.0000x reference)
import jax
import jax.numpy as jnp
from jax.experimental import pallas as pl
from jax.experimental.pallas import tpu as pltpu

_BB = 4096


def _copy_kernel(bs_ref, out_ref):
    out_ref[...] = bs_ref[...]


def kernel(board_state, premises, heads, bias):
    b = board_state.shape[0]
    rows = b * 27 // 128
    bs2 = board_state.reshape(rows, 128)
    out2 = pl.pallas_call(
        _copy_kernel,
        grid=(rows // _BB,),
        in_specs=[pl.BlockSpec((_BB, 128), lambda i: (i, 0))],
        out_specs=pl.BlockSpec((_BB, 128), lambda i: (i, 0)),
        out_shape=jax.ShapeDtypeStruct((rows, 128), jnp.float32),
        compiler_params=pltpu.CompilerParams(
            dimension_semantics=("parallel",),
        ),
    )(bs2)
    return out2.reshape(b, 9, 3)


# speedup vs baseline: 53.3044x; 53.3044x over previous
import jax
import jax.numpy as jnp
from jax import lax
from jax.experimental import pallas as pl
from jax.experimental.pallas import tpu as pltpu

_NUM_PROPS = 9
_NUM_RULES = 8
_NUM_PREMISES = 2
_OUT_DIM = 27
_RP = _NUM_RULES * _NUM_PREMISES
_SIM_ROWS = _NUM_PROPS * _RP
_BB = 2048


def _block_kernel(bs_ref, p0_ref, p1_ref, headst_ref, bias_ref, out_ref):
    k_iota = lax.broadcasted_iota(jnp.int32, (_SIM_ROWS, _OUT_DIM), 1)
    n_iota = lax.broadcasted_iota(jnp.int32, (_SIM_ROWS, _OUT_DIM), 0)
    c = k_iota % 3
    i_k = k_iota // 3
    i_n = n_iota // _RP
    player = jnp.where(c == 1, 1.0, jnp.where(c == 2, -1.0, 0.0))
    pos = (i_k.astype(jnp.float32) - 4.0) * 0.25
    d0 = player - p0_ref[...]
    d1 = pos - p1_ref[...]
    w = jnp.exp(-(d0 * d0 + d1 * d1))
    w = jnp.where(i_n == i_k, w, 0.0)

    bs27 = bs_ref[...].reshape(_OUT_DIM, bs_ref.shape[2])
    sim_t = lax.dot_general(w, bs27, (((1,), (0,)), ((), ())),
                            preferred_element_type=jnp.float32)

    sat = sim_t[0:_RP, :]
    for i in range(1, _NUM_PROPS):
        sat = jnp.maximum(sat, sim_t[i * _RP:(i + 1) * _RP, :])

    act = sat[0:_NUM_RULES, :] * sat[_NUM_RULES:_RP, :]

    out = lax.dot_general(headst_ref[...], act, (((1,), (0,)), ((), ())),
                          preferred_element_type=jnp.float32)
    out = out + bias_ref[...]
    out_ref[...] = out.reshape(_NUM_PROPS, 3, bs_ref.shape[2])


def kernel(board_state, premises, heads, bias):
    b = board_state.shape[0]
    bs_t = board_state.transpose(1, 2, 0)
    prem_pr = premises.transpose(1, 0, 2).reshape(_RP, _NUM_PREMISES)
    p0b = jnp.broadcast_to(jnp.tile(prem_pr[:, 0], _NUM_PROPS)[:, None],
                           (_SIM_ROWS, _OUT_DIM))
    p1b = jnp.broadcast_to(jnp.tile(prem_pr[:, 1], _NUM_PROPS)[:, None],
                           (_SIM_ROWS, _OUT_DIM))
    heads_t = heads.T
    bias_col = bias.reshape(_OUT_DIM, 1)

    grid = (b // _BB,)
    out_t = pl.pallas_call(
        _block_kernel,
        grid=grid,
        in_specs=[
            pl.BlockSpec((_NUM_PROPS, 3, _BB), lambda i: (0, 0, i)),
            pl.BlockSpec((_SIM_ROWS, _OUT_DIM), lambda i: (0, 0)),
            pl.BlockSpec((_SIM_ROWS, _OUT_DIM), lambda i: (0, 0)),
            pl.BlockSpec((_OUT_DIM, _NUM_RULES), lambda i: (0, 0)),
            pl.BlockSpec((_OUT_DIM, 1), lambda i: (0, 0)),
        ],
        out_specs=pl.BlockSpec((_NUM_PROPS, 3, _BB), lambda i: (0, 0, i)),
        out_shape=jax.ShapeDtypeStruct((_NUM_PROPS, 3, b), jnp.float32),
        compiler_params=pltpu.CompilerParams(
            dimension_semantics=("parallel",),
        ),
    )(bs_t, p0b, p1b, heads_t, bias_col)
    return out_t.transpose(2, 0, 1)


# speedup vs baseline: 82.8028x; 1.5534x over previous
import jax
import jax.numpy as jnp
from jax import lax
from jax.experimental import pallas as pl
from jax.experimental.pallas import tpu as pltpu

_NUM_PROPS = 9
_NUM_RULES = 8
_NUM_PREMISES = 2
_OUT_DIM = 27
_RP = _NUM_RULES * _NUM_PREMISES
_SIM_ROWS = _NUM_PROPS * _RP
_BB = 4096


def _block_kernel(bs_ref, p0_ref, p1_ref, headst_ref, bias_ref, out_ref):
    k_iota = lax.broadcasted_iota(jnp.int32, (_SIM_ROWS, _OUT_DIM), 1)
    n_iota = lax.broadcasted_iota(jnp.int32, (_SIM_ROWS, _OUT_DIM), 0)
    c = k_iota % 3
    i_k = k_iota // 3
    i_n = n_iota // _RP
    player = jnp.where(c == 1, 1.0, jnp.where(c == 2, -1.0, 0.0))
    pos = (i_k.astype(jnp.float32) - 4.0) * 0.25
    d0 = player - p0_ref[...]
    d1 = pos - p1_ref[...]
    w = jnp.exp(-(d0 * d0 + d1 * d1))
    w = jnp.where(i_n == i_k, w, 0.0)

    bs27 = bs_ref[...].reshape(_OUT_DIM, bs_ref.shape[2])
    sim_t = lax.dot_general(w.astype(jnp.bfloat16),
                            bs27.astype(jnp.bfloat16),
                            (((1,), (0,)), ((), ())),
                            preferred_element_type=jnp.float32)

    sat = sim_t[0:_RP, :]
    for i in range(1, _NUM_PROPS):
        sat = jnp.maximum(sat, sim_t[i * _RP:(i + 1) * _RP, :])

    act = sat[0:_NUM_RULES, :] * sat[_NUM_RULES:_RP, :]

    out = lax.dot_general(headst_ref[...], act, (((1,), (0,)), ((), ())),
                          preferred_element_type=jnp.float32)
    out = out + bias_ref[...]
    out_ref[...] = out.reshape(_NUM_PROPS, 3, bs_ref.shape[2])


def kernel(board_state, premises, heads, bias):
    b = board_state.shape[0]
    bs_t = board_state.transpose(1, 2, 0)
    prem_pr = premises.transpose(1, 0, 2).reshape(_RP, _NUM_PREMISES)
    p0b = jnp.broadcast_to(jnp.tile(prem_pr[:, 0], _NUM_PROPS)[:, None],
                           (_SIM_ROWS, _OUT_DIM))
    p1b = jnp.broadcast_to(jnp.tile(prem_pr[:, 1], _NUM_PROPS)[:, None],
                           (_SIM_ROWS, _OUT_DIM))
    heads_t = heads.T
    bias_col = bias.reshape(_OUT_DIM, 1)

    grid = (b // _BB,)
    out_t = pl.pallas_call(
        _block_kernel,
        grid=grid,
        in_specs=[
            pl.BlockSpec((_NUM_PROPS, 3, _BB), lambda i: (0, 0, i)),
            pl.BlockSpec((_SIM_ROWS, _OUT_DIM), lambda i: (0, 0)),
            pl.BlockSpec((_SIM_ROWS, _OUT_DIM), lambda i: (0, 0)),
            pl.BlockSpec((_OUT_DIM, _NUM_RULES), lambda i: (0, 0)),
            pl.BlockSpec((_OUT_DIM, 1), lambda i: (0, 0)),
        ],
        out_specs=pl.BlockSpec((_NUM_PROPS, 3, _BB), lambda i: (0, 0, i)),
        out_shape=jax.ShapeDtypeStruct((_NUM_PROPS, 3, b), jnp.float32),
        compiler_params=pltpu.CompilerParams(
            dimension_semantics=("parallel",),
        ),
    )(bs_t, p0b, p1b, heads_t, bias_col)
    return out_t.transpose(2, 0, 1)


# speedup vs baseline: 84.7502x; 1.0235x over previous
import jax
import jax.numpy as jnp
from jax import lax
from jax.experimental import pallas as pl
from jax.experimental.pallas import tpu as pltpu

_NUM_PROPS = 9
_NUM_RULES = 8
_NUM_PREMISES = 2
_OUT_DIM = 27
_RP = _NUM_RULES * _NUM_PREMISES
_SIM_ROWS = _NUM_PROPS * _RP
_K36 = _NUM_PROPS * 4
_BB = 4096


def _block_kernel(bs_ref, p0_ref, p1_ref, headst_ref, bias_ref, out_ref):
    bb = bs_ref.shape[2]
    k_iota = lax.broadcasted_iota(jnp.int32, (_SIM_ROWS, _K36), 1)
    n_iota = lax.broadcasted_iota(jnp.int32, (_SIM_ROWS, _K36), 0)
    c = k_iota % 4
    i_k = k_iota // 4
    i_n = n_iota // _RP
    player = jnp.where(c == 1, 1.0, jnp.where(c == 2, -1.0, 0.0))
    pos = (i_k.astype(jnp.float32) - 4.0) * 0.25
    d0 = player - p0_ref[...]
    d1 = pos - p1_ref[...]
    w = jnp.exp(-(d0 * d0 + d1 * d1))
    w = jnp.where((i_n == i_k) & (c < 3), w, 0.0)

    bs36 = jnp.pad(bs_ref[...], ((0, 0), (0, 1), (0, 0))).reshape(_K36, bb)
    sim_t = lax.dot_general(w.astype(jnp.bfloat16),
                            bs36.astype(jnp.bfloat16),
                            (((1,), (0,)), ((), ())),
                            preferred_element_type=jnp.float32)

    sat = sim_t[0:_RP, :]
    for i in range(1, _NUM_PROPS):
        sat = jnp.maximum(sat, sim_t[i * _RP:(i + 1) * _RP, :])

    act = sat[0:_NUM_RULES, :] * sat[_NUM_RULES:_RP, :]

    out36 = lax.dot_general(headst_ref[...], act, (((1,), (0,)), ((), ())),
                            preferred_element_type=jnp.float32)
    out36 = out36 + bias_ref[...]
    out_ref[...] = out36.reshape(_NUM_PROPS, 4, bb)[:, 0:3, :]


def kernel(board_state, premises, heads, bias):
    b = board_state.shape[0]
    bs_t = board_state.transpose(1, 2, 0)
    prem_pr = premises.transpose(1, 0, 2).reshape(_RP, _NUM_PREMISES)
    p0b = jnp.broadcast_to(jnp.tile(prem_pr[:, 0], _NUM_PROPS)[:, None],
                           (_SIM_ROWS, _K36))
    p1b = jnp.broadcast_to(jnp.tile(prem_pr[:, 1], _NUM_PROPS)[:, None],
                           (_SIM_ROWS, _K36))
    heads_t4 = jnp.pad(heads.T.reshape(_NUM_PROPS, 3, _NUM_RULES),
                       ((0, 0), (0, 1), (0, 0))).reshape(_K36, _NUM_RULES)
    bias4 = jnp.pad(bias.reshape(_NUM_PROPS, 3),
                    ((0, 0), (0, 1))).reshape(_K36, 1)

    grid = (b // _BB,)
    out_t = pl.pallas_call(
        _block_kernel,
        grid=grid,
        in_specs=[
            pl.BlockSpec((_NUM_PROPS, 3, _BB), lambda i: (0, 0, i)),
            pl.BlockSpec((_SIM_ROWS, _K36), lambda i: (0, 0)),
            pl.BlockSpec((_SIM_ROWS, _K36), lambda i: (0, 0)),
            pl.BlockSpec((_K36, _NUM_RULES), lambda i: (0, 0)),
            pl.BlockSpec((_K36, 1), lambda i: (0, 0)),
        ],
        out_specs=pl.BlockSpec((_NUM_PROPS, 3, _BB), lambda i: (0, 0, i)),
        out_shape=jax.ShapeDtypeStruct((_NUM_PROPS, 3, b), jnp.float32),
        compiler_params=pltpu.CompilerParams(
            dimension_semantics=("parallel",),
        ),
    )(bs_t, p0b, p1b, heads_t4, bias4)
    return out_t.transpose(2, 0, 1)


# speedup vs baseline: 116.5828x; 1.3756x over previous
import jax
import jax.numpy as jnp
from jax import lax
from jax.experimental import pallas as pl
from jax.experimental.pallas import tpu as pltpu

_NUM_PROPS = 9
_NUM_RULES = 8
_NUM_PREMISES = 2
_OUT_DIM = 27
_RP = _NUM_RULES * _NUM_PREMISES
_SIM_ROWS = _NUM_PROPS * _RP
_K36 = _NUM_PROPS * 4
_BB = 8192


def _block_kernel(bs_ref, p0_ref, p1_ref, headst_ref, bias_ref, out_ref):
    bb = bs_ref.shape[2]
    k_iota = lax.broadcasted_iota(jnp.int32, (_SIM_ROWS, _K36), 1)
    n_iota = lax.broadcasted_iota(jnp.int32, (_SIM_ROWS, _K36), 0)
    c = k_iota % 4
    i_k = k_iota // 4
    i_n = n_iota // _RP
    player = jnp.where(c == 1, 1.0, jnp.where(c == 2, -1.0, 0.0))
    pos = (i_k.astype(jnp.float32) - 4.0) * 0.25
    d0 = player - p0_ref[...]
    d1 = pos - p1_ref[...]
    w = jnp.exp(-(d0 * d0 + d1 * d1))
    w = jnp.where((i_n == i_k) & (c < 3), w, 0.0)

    bs36 = jnp.pad(bs_ref[...], ((0, 0), (0, 1), (0, 0))).reshape(_K36, bb)
    sim_t = lax.dot_general(w.astype(jnp.bfloat16),
                            bs36.astype(jnp.bfloat16),
                            (((1,), (0,)), ((), ())),
                            preferred_element_type=jnp.float32)

    sat = sim_t[0:_RP, :]
    for i in range(1, _NUM_PROPS):
        sat = jnp.maximum(sat, sim_t[i * _RP:(i + 1) * _RP, :])

    act = sat[0:_NUM_RULES, :] * sat[_NUM_RULES:_RP, :]

    out36 = lax.dot_general(headst_ref[...], act, (((1,), (0,)), ((), ())),
                            preferred_element_type=jnp.float32)
    out36 = out36 + bias_ref[...]
    out_ref[...] = out36.reshape(_NUM_PROPS, 4, bb)[:, 0:3, :]


def kernel(board_state, premises, heads, bias):
    b = board_state.shape[0]
    bs_t = board_state.transpose(1, 2, 0)
    prem_pr = premises.transpose(1, 0, 2).reshape(_RP, _NUM_PREMISES)
    p0b = jnp.broadcast_to(jnp.tile(prem_pr[:, 0], _NUM_PROPS)[:, None],
                           (_SIM_ROWS, _K36))
    p1b = jnp.broadcast_to(jnp.tile(prem_pr[:, 1], _NUM_PROPS)[:, None],
                           (_SIM_ROWS, _K36))
    heads_t4 = jnp.pad(heads.T.reshape(_NUM_PROPS, 3, _NUM_RULES),
                       ((0, 0), (0, 1), (0, 0))).reshape(_K36, _NUM_RULES)
    bias4 = jnp.pad(bias.reshape(_NUM_PROPS, 3),
                    ((0, 0), (0, 1))).reshape(_K36, 1)

    grid = (b // _BB,)
    out_t = pl.pallas_call(
        _block_kernel,
        grid=grid,
        in_specs=[
            pl.BlockSpec((_NUM_PROPS, 3, _BB), lambda i: (0, 0, i)),
            pl.BlockSpec((_SIM_ROWS, _K36), lambda i: (0, 0)),
            pl.BlockSpec((_SIM_ROWS, _K36), lambda i: (0, 0)),
            pl.BlockSpec((_K36, _NUM_RULES), lambda i: (0, 0)),
            pl.BlockSpec((_K36, 1), lambda i: (0, 0)),
        ],
        out_specs=pl.BlockSpec((_NUM_PROPS, 3, _BB), lambda i: (0, 0, i)),
        out_shape=jax.ShapeDtypeStruct((_NUM_PROPS, 3, b), jnp.float32),
        compiler_params=pltpu.CompilerParams(
            dimension_semantics=("parallel",),
        ),
    )(bs_t, p0b, p1b, heads_t4, bias4)
    return out_t.transpose(2, 0, 1)


# speedup vs baseline: 145.2251x; 1.2457x over previous
import jax
import jax.numpy as jnp
from jax import lax
from jax.experimental import pallas as pl
from jax.experimental.pallas import tpu as pltpu

_NUM_PROPS = 9
_NUM_RULES = 8
_NUM_PREMISES = 2
_OUT_DIM = 27
_RP = _NUM_RULES * _NUM_PREMISES
_SIM_ROWS = _NUM_PROPS * _RP
_K36 = _NUM_PROPS * 4
_BB = 16384


def _block_kernel(bs_ref, p0_ref, p1_ref, headst_ref, bias_ref, out_ref):
    bb = bs_ref.shape[2]
    k_iota = lax.broadcasted_iota(jnp.int32, (_SIM_ROWS, _K36), 1)
    n_iota = lax.broadcasted_iota(jnp.int32, (_SIM_ROWS, _K36), 0)
    c = k_iota % 4
    i_k = k_iota // 4
    i_n = n_iota // _RP
    player = jnp.where(c == 1, 1.0, jnp.where(c == 2, -1.0, 0.0))
    pos = (i_k.astype(jnp.float32) - 4.0) * 0.25
    d0 = player - p0_ref[...]
    d1 = pos - p1_ref[...]
    w = jnp.exp(-(d0 * d0 + d1 * d1))
    w = jnp.where((i_n == i_k) & (c < 3), w, 0.0)

    bs36 = jnp.pad(bs_ref[...], ((0, 0), (0, 1), (0, 0))).reshape(_K36, bb)
    sim_t = lax.dot_general(w.astype(jnp.bfloat16),
                            bs36.astype(jnp.bfloat16),
                            (((1,), (0,)), ((), ())),
                            preferred_element_type=jnp.float32)

    sat = sim_t[0:_RP, :]
    for i in range(1, _NUM_PROPS):
        sat = jnp.maximum(sat, sim_t[i * _RP:(i + 1) * _RP, :])

    act = sat[0:_NUM_RULES, :] * sat[_NUM_RULES:_RP, :]

    out36 = lax.dot_general(headst_ref[...], act, (((1,), (0,)), ((), ())),
                            preferred_element_type=jnp.float32)
    out36 = out36 + bias_ref[...]
    out_ref[...] = out36.reshape(_NUM_PROPS, 4, bb)[:, 0:3, :]


def kernel(board_state, premises, heads, bias):
    b = board_state.shape[0]
    bs_t = board_state.transpose(1, 2, 0)
    prem_pr = premises.transpose(1, 0, 2).reshape(_RP, _NUM_PREMISES)
    p0b = jnp.broadcast_to(jnp.tile(prem_pr[:, 0], _NUM_PROPS)[:, None],
                           (_SIM_ROWS, _K36))
    p1b = jnp.broadcast_to(jnp.tile(prem_pr[:, 1], _NUM_PROPS)[:, None],
                           (_SIM_ROWS, _K36))
    heads_t4 = jnp.pad(heads.T.reshape(_NUM_PROPS, 3, _NUM_RULES),
                       ((0, 0), (0, 1), (0, 0))).reshape(_K36, _NUM_RULES)
    bias4 = jnp.pad(bias.reshape(_NUM_PROPS, 3),
                    ((0, 0), (0, 1))).reshape(_K36, 1)

    grid = (b // _BB,)
    out_t = pl.pallas_call(
        _block_kernel,
        grid=grid,
        in_specs=[
            pl.BlockSpec((_NUM_PROPS, 3, _BB), lambda i: (0, 0, i)),
            pl.BlockSpec((_SIM_ROWS, _K36), lambda i: (0, 0)),
            pl.BlockSpec((_SIM_ROWS, _K36), lambda i: (0, 0)),
            pl.BlockSpec((_K36, _NUM_RULES), lambda i: (0, 0)),
            pl.BlockSpec((_K36, 1), lambda i: (0, 0)),
        ],
        out_specs=pl.BlockSpec((_NUM_PROPS, 3, _BB), lambda i: (0, 0, i)),
        out_shape=jax.ShapeDtypeStruct((_NUM_PROPS, 3, b), jnp.float32),
        compiler_params=pltpu.CompilerParams(
            dimension_semantics=("parallel",),
        ),
    )(bs_t, p0b, p1b, heads_t4, bias4)
    return out_t.transpose(2, 0, 1)


# speedup vs baseline: 163.0351x; 1.1226x over previous
import jax
import jax.numpy as jnp
from jax import lax
from jax.experimental import pallas as pl
from jax.experimental.pallas import tpu as pltpu

_NUM_PROPS = 9
_NUM_RULES = 8
_NUM_PREMISES = 2
_OUT_DIM = 27
_RP = _NUM_RULES * _NUM_PREMISES
_SIM_ROWS = _NUM_PROPS * _RP
_K36 = _NUM_PROPS * 4
_BB = 32768


def _block_kernel(bs_ref, p0_ref, p1_ref, headst_ref, bias_ref, out_ref):
    bb = bs_ref.shape[2]
    k_iota = lax.broadcasted_iota(jnp.int32, (_SIM_ROWS, _K36), 1)
    n_iota = lax.broadcasted_iota(jnp.int32, (_SIM_ROWS, _K36), 0)
    c = k_iota % 4
    i_k = k_iota // 4
    i_n = n_iota // _RP
    player = jnp.where(c == 1, 1.0, jnp.where(c == 2, -1.0, 0.0))
    pos = (i_k.astype(jnp.float32) - 4.0) * 0.25
    d0 = player - p0_ref[...]
    d1 = pos - p1_ref[...]
    w = jnp.exp(-(d0 * d0 + d1 * d1))
    w = jnp.where((i_n == i_k) & (c < 3), w, 0.0)

    bs36 = jnp.pad(bs_ref[...], ((0, 0), (0, 1), (0, 0))).reshape(_K36, bb)
    sim_t = lax.dot_general(w.astype(jnp.bfloat16),
                            bs36.astype(jnp.bfloat16),
                            (((1,), (0,)), ((), ())),
                            preferred_element_type=jnp.float32)

    sat = sim_t[0:_RP, :]
    for i in range(1, _NUM_PROPS):
        sat = jnp.maximum(sat, sim_t[i * _RP:(i + 1) * _RP, :])

    act = sat[0:_NUM_RULES, :] * sat[_NUM_RULES:_RP, :]

    out36 = lax.dot_general(headst_ref[...], act, (((1,), (0,)), ((), ())),
                            preferred_element_type=jnp.float32)
    out36 = out36 + bias_ref[...]
    out_ref[...] = out36.reshape(_NUM_PROPS, 4, bb)[:, 0:3, :]


def kernel(board_state, premises, heads, bias):
    b = board_state.shape[0]
    bs_t = board_state.transpose(1, 2, 0)
    prem_pr = premises.transpose(1, 0, 2).reshape(_RP, _NUM_PREMISES)
    p0b = jnp.broadcast_to(jnp.tile(prem_pr[:, 0], _NUM_PROPS)[:, None],
                           (_SIM_ROWS, _K36))
    p1b = jnp.broadcast_to(jnp.tile(prem_pr[:, 1], _NUM_PROPS)[:, None],
                           (_SIM_ROWS, _K36))
    heads_t4 = jnp.pad(heads.T.reshape(_NUM_PROPS, 3, _NUM_RULES),
                       ((0, 0), (0, 1), (0, 0))).reshape(_K36, _NUM_RULES)
    bias4 = jnp.pad(bias.reshape(_NUM_PROPS, 3),
                    ((0, 0), (0, 1))).reshape(_K36, 1)

    grid = (b // _BB,)
    out_t = pl.pallas_call(
        _block_kernel,
        grid=grid,
        in_specs=[
            pl.BlockSpec((_NUM_PROPS, 3, _BB), lambda i: (0, 0, i)),
            pl.BlockSpec((_SIM_ROWS, _K36), lambda i: (0, 0)),
            pl.BlockSpec((_SIM_ROWS, _K36), lambda i: (0, 0)),
            pl.BlockSpec((_K36, _NUM_RULES), lambda i: (0, 0)),
            pl.BlockSpec((_K36, 1), lambda i: (0, 0)),
        ],
        out_specs=pl.BlockSpec((_NUM_PROPS, 3, _BB), lambda i: (0, 0, i)),
        out_shape=jax.ShapeDtypeStruct((_NUM_PROPS, 3, b), jnp.float32),
        compiler_params=pltpu.CompilerParams(
            dimension_semantics=("parallel",),
        ),
    )(bs_t, p0b, p1b, heads_t4, bias4)
    return out_t.transpose(2, 0, 1)
